# Initial kernel scaffold; baseline (speedup 1.0000x reference)
import jax
import jax.numpy as jnp
from jax.experimental import pallas as pl


def kernel(atom_features, atom_positions, block_features, block_id, centers, widths, Wg, bg, Wq, bq, Wk, bk, Wv, bv, Wc1, bc1, Wc2, bc2, Wf1, bf1, Wf2, bf2, ln1_g, ln1_b, ln2_g, ln2_b):
    raise NotImplementedError("write your pallas kernel here")


# speedup vs baseline: 7.9497x; 7.9497x over previous
import functools
import math

import jax
import jax.numpy as jnp
from jax.experimental import pallas as pl
from jax.experimental.pallas import tpu as pltpu

H = 128
NEG = -1e30


def _centroid_kernel(ids_ref, pos_ref, cent_ref, *, nb, n):
    idsv = ids_ref[0, 0, :]
    blk = jax.lax.broadcasted_iota(jnp.int32, (nb, n), 0)
    O = (blk == idsv[None, :]).astype(jnp.float32)
    sums = jnp.dot(O, pos_ref[...], preferred_element_type=jnp.float32)
    cnt = jnp.sum(O, axis=1, keepdims=True)
    cent_ref[...] = sums / jnp.maximum(cnt, 1.0)


def _attn_kernel(ids_ref, feat_ref, pos_ref, cent_ref, bfeat_ref,
                 centers_ref, inv2w2_ref, Wg_ref, bg_ref,
                 Wq_ref, bq_ref, Wkt_ref, Wkb_ref, bk_ref,
                 Wvt_ref, Wvb_ref, bv_ref,
                 Wc1_ref, bc1_ref, Wc2_ref, bc2_ref,
                 h_ref, q_scr, m_scr, s_scr, c_scr, *, nb, c, nsteps):
    i = pl.program_id(0)

    @pl.when(i == 0)
    def _init():
        q_scr[...] = (jnp.dot(bfeat_ref[...], Wq_ref[...],
                              preferred_element_type=jnp.float32)
                      + bq_ref[...])
        m_scr[...] = jnp.full((nb, 1), NEG, jnp.float32)
        s_scr[...] = jnp.zeros((nb, 1), jnp.float32)
        c_scr[...] = jnp.zeros((nb, H), jnp.float32)

    idsv = ids_ref[0, 0, :]
    O = (jax.lax.broadcasted_iota(jnp.int32, (nb, c), 0)
         == idsv[None, :])
    Of = O.astype(jnp.float32)

    cent_g = jax.lax.dot_general(Of, cent_ref[...],
                                 (((0,), (0,)), ((), ())),
                                 preferred_element_type=jnp.float32)
    rel = pos_ref[...] - cent_g
    d = jnp.sqrt(jnp.sum(rel * rel, axis=1, keepdims=True))
    rbf = jnp.exp(-jnp.square(d - centers_ref[...]) * inv2w2_ref[...])
    geom = (jnp.dot(rbf, Wg_ref[...], preferred_element_type=jnp.float32)
            + bg_ref[...])

    feats = feat_ref[...]
    K = (jnp.dot(feats, Wkt_ref[...], preferred_element_type=jnp.float32)
         + jnp.dot(geom, Wkb_ref[...], preferred_element_type=jnp.float32)
         + bk_ref[...])
    V = (jnp.dot(feats, Wvt_ref[...], preferred_element_type=jnp.float32)
         + jnp.dot(geom, Wvb_ref[...], preferred_element_type=jnp.float32)
         + bv_ref[...])

    S = jax.lax.dot_general(q_scr[...], K, (((1,), (1,)), ((), ())),
                            preferred_element_type=jnp.float32)
    S = S * (1.0 / math.sqrt(H))
    Sm = jnp.where(O, S, NEG)
    m_old = m_scr[...]
    m_new = jnp.maximum(m_old, jnp.max(Sm, axis=1, keepdims=True))
    alpha = jnp.exp(m_old - m_new)
    e = jnp.where(O, jnp.exp(S - m_new), 0.0)
    m_scr[...] = m_new
    s_scr[...] = s_scr[...] * alpha + jnp.sum(e, axis=1, keepdims=True)
    c_scr[...] = (c_scr[...] * alpha
                  + jnp.dot(e, V, preferred_element_type=jnp.float32))

    @pl.when(i == nsteps - 1)
    def _finish():
        s = s_scr[...]
        ctx = c_scr[...] / jnp.where(s > 0.0, s, 1.0)
        h1 = jnp.maximum(
            jnp.dot(ctx, Wc1_ref[...], preferred_element_type=jnp.float32)
            + bc1_ref[...], 0.0)
        h_ref[...] = (jnp.dot(h1, Wc2_ref[...],
                              preferred_element_type=jnp.float32)
                      + bc2_ref[...])


def _ln(x, g, b, eps=1e-5):
    m = jnp.mean(x, axis=-1, keepdims=True)
    xc = x - m
    v = jnp.mean(xc * xc, axis=-1, keepdims=True)
    return xc * jax.lax.rsqrt(v + eps) * g + b


def _out_kernel(ids_ref, feat_ref, h_ref,
                Wf1_ref, bf1_ref, Wf2_ref, bf2_ref,
                ln1g_ref, ln1b_ref, ln2g_ref, ln2b_ref,
                out_ref, *, nb, c):
    idsv = ids_ref[0, 0, :]
    OT = (idsv[:, None]
          == jax.lax.broadcasted_iota(jnp.int32, (c, nb), 1)).astype(jnp.float32)
    upd = jnp.dot(OT, h_ref[...], preferred_element_type=jnp.float32)
    u1 = _ln(feat_ref[...] + upd, ln1g_ref[...], ln1b_ref[...])
    f1 = jnp.maximum(
        jnp.dot(u1, Wf1_ref[...], preferred_element_type=jnp.float32)
        + bf1_ref[...], 0.0)
    ffn = (jnp.dot(f1, Wf2_ref[...], preferred_element_type=jnp.float32)
           + bf2_ref[...])
    out_ref[...] = _ln(u1 + ffn, ln2g_ref[...], ln2b_ref[...])


def kernel(atom_features, atom_positions, block_features, block_id,
           centers, widths, Wg, bg, Wq, bq, Wk, bk, Wv, bv,
           Wc1, bc1, Wc2, bc2, Wf1, bf1, Wf2, bf2,
           ln1_g, ln1_b, ln2_g, ln2_b):
    n, h = atom_features.shape
    nb = block_features.shape[0]
    rbf_dim = centers.shape[0]
    hq = Wg.shape[1]
    C = 2048
    nsteps = n // C

    ids = block_id.astype(jnp.int32)
    ids_full = ids.reshape(1, 1, n)
    ids_chunked = ids.reshape(nsteps, 1, C)

    centers2 = centers.reshape(1, rbf_dim).astype(jnp.float32)
    inv2w2 = (1.0 / (2.0 * jnp.square(widths))).reshape(1, rbf_dim)
    row = lambda v: v.reshape(1, -1)

    centroids = pl.pallas_call(
        functools.partial(_centroid_kernel, nb=nb, n=n),
        out_shape=jax.ShapeDtypeStruct((nb, 3), jnp.float32),
    )(ids_full, atom_positions)

    Wk_top, Wk_bot = Wk[:h], Wk[h:]
    Wv_top, Wv_bot = Wv[:h], Wv[h:]

    full = lambda shape: pl.BlockSpec(shape, lambda i: (0,) * len(shape))
    h_blocks = pl.pallas_call(
        functools.partial(_attn_kernel, nb=nb, c=C, nsteps=nsteps),
        grid=(nsteps,),
        in_specs=[
            pl.BlockSpec((1, 1, C), lambda i: (i, 0, 0)),
            pl.BlockSpec((C, h), lambda i: (i, 0)),
            pl.BlockSpec((C, 3), lambda i: (i, 0)),
            full((nb, 3)),
            full((nb, h)),
            full((1, rbf_dim)),
            full((1, rbf_dim)),
            full((rbf_dim, hq)),
            full((1, hq)),
            full((h, h)),
            full((1, h)),
            full((h, h)),
            full((hq, h)),
            full((1, h)),
            full((h, h)),
            full((hq, h)),
            full((1, h)),
            full((h, h)),
            full((1, h)),
            full((h, h)),
            full((1, h)),
        ],
        out_specs=full((nb, h)),
        out_shape=jax.ShapeDtypeStruct((nb, h), jnp.float32),
        scratch_shapes=[
            pltpu.VMEM((nb, h), jnp.float32),
            pltpu.VMEM((nb, 1), jnp.float32),
            pltpu.VMEM((nb, 1), jnp.float32),
            pltpu.VMEM((nb, h), jnp.float32),
        ],
    )(ids_chunked, atom_features, atom_positions, centroids, block_features,
      centers2, inv2w2, Wg, row(bg), Wq, row(bq), Wk_top, Wk_bot, row(bk),
      Wv_top, Wv_bot, row(bv), Wc1, row(bc1), Wc2, row(bc2))

    out = pl.pallas_call(
        functools.partial(_out_kernel, nb=nb, c=C),
        grid=(nsteps,),
        in_specs=[
            pl.BlockSpec((1, 1, C), lambda i: (i, 0, 0)),
            pl.BlockSpec((C, h), lambda i: (i, 0)),
            full((nb, h)),
            full((h, 2 * h)),
            full((1, 2 * h)),
            full((2 * h, h)),
            full((1, h)),
            full((1, h)),
            full((1, h)),
            full((1, h)),
            full((1, h)),
        ],
        out_specs=pl.BlockSpec((C, h), lambda i: (i, 0)),
        out_shape=jax.ShapeDtypeStruct((n, h), jnp.float32),
    )(ids_chunked, atom_features, h_blocks,
      Wf1, row(bf1), Wf2, row(bf2),
      row(ln1_g), row(ln1_b), row(ln2_g), row(ln2_b))

    return out


# speedup vs baseline: 8.0764x; 1.0159x over previous
import functools
import math

import jax
import jax.numpy as jnp
from jax.experimental import pallas as pl
from jax.experimental.pallas import tpu as pltpu

H = 128
NEG = -1e30


def _centroid_kernel(ids_ref, pos_ref, cent_ref, *, nb, n):
    idsv = ids_ref[0, 0, :]
    blk = jax.lax.broadcasted_iota(jnp.int32, (nb, n), 0)
    O = (blk == idsv[None, :]).astype(jnp.float32)
    sums = jnp.dot(O, pos_ref[...], preferred_element_type=jnp.float32)
    cnt = jnp.sum(O, axis=1, keepdims=True)
    cent_ref[...] = sums / jnp.maximum(cnt, 1.0)


def _attn_kernel(ids_ref, feat_ref, pos_ref, cent_ref, bfeat_ref,
                 centers_ref, inv2w2_ref, Wg_ref, bg_ref,
                 Wq_ref, bq_ref, Wkt_ref, Wkb_ref, bk_ref,
                 Wvt_ref, Wvb_ref, bv_ref,
                 Wc1_ref, bc1_ref, Wc2_ref, bc2_ref,
                 h_ref, q_scr, m_scr, s_scr, c_scr, *, nb, c, nsteps):
    i = pl.program_id(0)

    @pl.when(i == 0)
    def _init():
        q_scr[...] = (jnp.dot(bfeat_ref[...], Wq_ref[...],
                              preferred_element_type=jnp.float32)
                      + bq_ref[...])
        m_scr[...] = jnp.full((nb, 1), NEG, jnp.float32)
        s_scr[...] = jnp.zeros((nb, 1), jnp.float32)
        c_scr[...] = jnp.zeros((nb, H), jnp.float32)

    idsv = ids_ref[0, 0, :]
    O = (jax.lax.broadcasted_iota(jnp.int32, (nb, c), 0)
         == idsv[None, :])
    Of = O.astype(jnp.float32)

    cent_g = jax.lax.dot_general(Of, cent_ref[...],
                                 (((0,), (0,)), ((), ())),
                                 preferred_element_type=jnp.float32)
    rel = pos_ref[...] - cent_g
    d = jnp.sqrt(jnp.sum(rel * rel, axis=1, keepdims=True))
    rbf = jnp.exp(-jnp.square(d - centers_ref[...]) * inv2w2_ref[...])
    geom = (jnp.dot(rbf, Wg_ref[...], preferred_element_type=jnp.float32)
            + bg_ref[...])

    feats = feat_ref[...].astype(jnp.bfloat16)
    K = (jnp.dot(feats, Wkt_ref[...], preferred_element_type=jnp.float32)
         + jnp.dot(geom, Wkb_ref[...], preferred_element_type=jnp.float32)
         + bk_ref[...])
    V = (jnp.dot(feats, Wvt_ref[...], preferred_element_type=jnp.float32)
         + jnp.dot(geom, Wvb_ref[...], preferred_element_type=jnp.float32)
         + bv_ref[...])

    S = jax.lax.dot_general(q_scr[...], K, (((1,), (1,)), ((), ())),
                            preferred_element_type=jnp.float32)
    S = S * (1.0 / math.sqrt(H))
    Sm = jnp.where(O, S, NEG)
    m_old = m_scr[...]
    m_new = jnp.maximum(m_old, jnp.max(Sm, axis=1, keepdims=True))
    alpha = jnp.exp(m_old - m_new)
    e = jnp.where(O, jnp.exp(S - m_new), 0.0)
    m_scr[...] = m_new
    s_scr[...] = s_scr[...] * alpha + jnp.sum(e, axis=1, keepdims=True)
    c_scr[...] = (c_scr[...] * alpha
                  + jnp.dot(e, V, preferred_element_type=jnp.float32))

    @pl.when(i == nsteps - 1)
    def _finish():
        s = s_scr[...]
        ctx = c_scr[...] / jnp.where(s > 0.0, s, 1.0)
        h1 = jnp.maximum(
            jnp.dot(ctx, Wc1_ref[...], preferred_element_type=jnp.float32)
            + bc1_ref[...], 0.0)
        h_ref[...] = (jnp.dot(h1, Wc2_ref[...],
                              preferred_element_type=jnp.float32)
                      + bc2_ref[...])


def _ln(x, g, b, eps=1e-5):
    m = jnp.mean(x, axis=-1, keepdims=True)
    xc = x - m
    v = jnp.mean(xc * xc, axis=-1, keepdims=True)
    return xc * jax.lax.rsqrt(v + eps) * g + b


def _out_kernel(ids_ref, feat_ref, h_ref,
                Wf1_ref, bf1_ref, Wf2_ref, bf2_ref,
                ln1g_ref, ln1b_ref, ln2g_ref, ln2b_ref,
                out_ref, *, nb, c):
    idsv = ids_ref[0, 0, :]
    OT = (idsv[:, None]
          == jax.lax.broadcasted_iota(jnp.int32, (c, nb), 1)).astype(jnp.float32)
    upd = jnp.dot(OT, h_ref[...], preferred_element_type=jnp.float32)
    u1 = _ln(feat_ref[...] + upd, ln1g_ref[...], ln1b_ref[...])
    f1 = jnp.maximum(
        jnp.dot(u1.astype(jnp.bfloat16), Wf1_ref[...],
                preferred_element_type=jnp.float32)
        + bf1_ref[...], 0.0)
    ffn = (jnp.dot(f1.astype(jnp.bfloat16), Wf2_ref[...],
                   preferred_element_type=jnp.float32)
           + bf2_ref[...])
    out_ref[...] = _ln(u1 + ffn, ln2g_ref[...], ln2b_ref[...])


def kernel(atom_features, atom_positions, block_features, block_id,
           centers, widths, Wg, bg, Wq, bq, Wk, bk, Wv, bv,
           Wc1, bc1, Wc2, bc2, Wf1, bf1, Wf2, bf2,
           ln1_g, ln1_b, ln2_g, ln2_b):
    n, h = atom_features.shape
    nb = block_features.shape[0]
    rbf_dim = centers.shape[0]
    hq = Wg.shape[1]
    C = 2048
    nsteps = n // C

    ids = block_id.astype(jnp.int32)
    ids_full = ids.reshape(1, 1, n)
    ids_chunked = ids.reshape(nsteps, 1, C)

    centers2 = centers.reshape(1, rbf_dim).astype(jnp.float32)
    inv2w2 = (1.0 / (2.0 * jnp.square(widths))).reshape(1, rbf_dim)
    row = lambda v: v.reshape(1, -1)

    centroids = pl.pallas_call(
        functools.partial(_centroid_kernel, nb=nb, n=n),
        out_shape=jax.ShapeDtypeStruct((nb, 3), jnp.float32),
    )(ids_full, atom_positions)

    Wk_top, Wk_bot = Wk[:h].astype(jnp.bfloat16), Wk[h:]
    Wv_top, Wv_bot = Wv[:h].astype(jnp.bfloat16), Wv[h:]
    Wf1_bf = Wf1.astype(jnp.bfloat16)
    Wf2_bf = Wf2.astype(jnp.bfloat16)

    full = lambda shape: pl.BlockSpec(shape, lambda i: (0,) * len(shape))
    h_blocks = pl.pallas_call(
        functools.partial(_attn_kernel, nb=nb, c=C, nsteps=nsteps),
        grid=(nsteps,),
        in_specs=[
            pl.BlockSpec((1, 1, C), lambda i: (i, 0, 0)),
            pl.BlockSpec((C, h), lambda i: (i, 0)),
            pl.BlockSpec((C, 3), lambda i: (i, 0)),
            full((nb, 3)),
            full((nb, h)),
            full((1, rbf_dim)),
            full((1, rbf_dim)),
            full((rbf_dim, hq)),
            full((1, hq)),
            full((h, h)),
            full((1, h)),
            full((h, h)),
            full((hq, h)),
            full((1, h)),
            full((h, h)),
            full((hq, h)),
            full((1, h)),
            full((h, h)),
            full((1, h)),
            full((h, h)),
            full((1, h)),
        ],
        out_specs=full((nb, h)),
        out_shape=jax.ShapeDtypeStruct((nb, h), jnp.float32),
        scratch_shapes=[
            pltpu.VMEM((nb, h), jnp.float32),
            pltpu.VMEM((nb, 1), jnp.float32),
            pltpu.VMEM((nb, 1), jnp.float32),
            pltpu.VMEM((nb, h), jnp.float32),
        ],
    )(ids_chunked, atom_features, atom_positions, centroids, block_features,
      centers2, inv2w2, Wg, row(bg), Wq, row(bq), Wk_top, Wk_bot, row(bk),
      Wv_top, Wv_bot, row(bv), Wc1, row(bc1), Wc2, row(bc2))

    out = pl.pallas_call(
        functools.partial(_out_kernel, nb=nb, c=C),
        grid=(nsteps,),
        in_specs=[
            pl.BlockSpec((1, 1, C), lambda i: (i, 0, 0)),
            pl.BlockSpec((C, h), lambda i: (i, 0)),
            full((nb, h)),
            full((h, 2 * h)),
            full((1, 2 * h)),
            full((2 * h, h)),
            full((1, h)),
            full((1, h)),
            full((1, h)),
            full((1, h)),
            full((1, h)),
        ],
        out_specs=pl.BlockSpec((C, h), lambda i: (i, 0)),
        out_shape=jax.ShapeDtypeStruct((n, h), jnp.float32),
    )(ids_chunked, atom_features, h_blocks,
      Wf1_bf, row(bf1), Wf2_bf, row(bf2),
      row(ln1_g), row(ln1_b), row(ln2_g), row(ln2_b))

    return out


# speedup vs baseline: 10.2037x; 1.2634x over previous
import functools
import math

import jax
import jax.numpy as jnp
from jax.experimental import pallas as pl
from jax.experimental.pallas import tpu as pltpu

H = 128
NEG = -1e30


def _centroid_kernel(ids_ref, posT_ref, cent_ref, *, nb, n):
    idsv = ids_ref[0, 0, :]
    blk = jax.lax.broadcasted_iota(jnp.int32, (nb, n), 0)
    O = (blk == idsv[None, :]).astype(jnp.float32)
    sums = jax.lax.dot_general(posT_ref[...], O, (((1,), (1,)), ((), ())),
                               preferred_element_type=jnp.float32)
    cnt = jax.lax.dot_general(jnp.ones((1, n), jnp.float32), O,
                              (((1,), (1,)), ((), ())),
                              preferred_element_type=jnp.float32)
    cent_ref[...] = sums / jnp.maximum(cnt, 1.0)


def _attn_kernel(ids_ref, feat_ref, posT_ref, cent_ref, bfeat_ref,
                 centers_ref, inv2w2_ref, WgT_ref, bg_ref,
                 Wq_ref, bq_ref, Wkt_ref, Wkb_ref, bk_ref,
                 Wvt_ref, Wvb_ref, bv_ref,
                 Wc1_ref, bc1_ref, Wc2_ref, bc2_ref,
                 h_ref, q_scr, m_scr, s_scr, c_scr, *, nb, c, nsteps):
    i = pl.program_id(0)

    @pl.when(i == 0)
    def _init():
        q_scr[...] = (jnp.dot(bfeat_ref[...], Wq_ref[...],
                              preferred_element_type=jnp.float32)
                      + bq_ref[...])
        m_scr[...] = jnp.full((nb, 1), NEG, jnp.float32)
        s_scr[...] = jnp.zeros((nb, 1), jnp.float32)
        c_scr[...] = jnp.zeros((nb, H), jnp.float32)

    idsv = ids_ref[0, 0, :]
    O = (jax.lax.broadcasted_iota(jnp.int32, (nb, c), 0)
         == idsv[None, :])
    Of = O.astype(jnp.float32)

    cent_g = jax.lax.dot_general(cent_ref[...], Of, (((1,), (0,)), ((), ())),
                                 preferred_element_type=jnp.float32)
    rel = posT_ref[...] - cent_g
    d = jnp.sqrt(jnp.sum(rel * rel, axis=0, keepdims=True))
    rbfT = jnp.exp(-jnp.square(d - centers_ref[...]) * inv2w2_ref[...])
    geomT = (jnp.dot(WgT_ref[...], rbfT, preferred_element_type=jnp.float32)
             + bg_ref[...]).astype(jnp.bfloat16)

    feats = feat_ref[...].astype(jnp.bfloat16)
    K = (jnp.dot(feats, Wkt_ref[...], preferred_element_type=jnp.float32)
         + jax.lax.dot_general(geomT, Wkb_ref[...], (((0,), (0,)), ((), ())),
                               preferred_element_type=jnp.float32)
         + bk_ref[...])
    V = (jnp.dot(feats, Wvt_ref[...], preferred_element_type=jnp.float32)
         + jax.lax.dot_general(geomT, Wvb_ref[...], (((0,), (0,)), ((), ())),
                               preferred_element_type=jnp.float32)
         + bv_ref[...])

    S = jax.lax.dot_general(q_scr[...], K, (((1,), (1,)), ((), ())),
                            preferred_element_type=jnp.float32)
    S = S * (1.0 / math.sqrt(H))
    Sm = jnp.where(O, S, NEG)
    m_old = m_scr[...]
    m_new = jnp.maximum(m_old, jnp.max(Sm, axis=1, keepdims=True))
    alpha = jnp.exp(m_old - m_new)
    e = jnp.where(O, jnp.exp(S - m_new), 0.0)
    m_scr[...] = m_new
    s_scr[...] = s_scr[...] * alpha + jnp.sum(e, axis=1, keepdims=True)
    c_scr[...] = (c_scr[...] * alpha
                  + jnp.dot(e, V, preferred_element_type=jnp.float32))

    @pl.when(i == nsteps - 1)
    def _finish():
        s = s_scr[...]
        ctx = c_scr[...] / jnp.where(s > 0.0, s, 1.0)
        h1 = jnp.maximum(
            jnp.dot(ctx, Wc1_ref[...], preferred_element_type=jnp.float32)
            + bc1_ref[...], 0.0)
        h_ref[...] = (jnp.dot(h1, Wc2_ref[...],
                              preferred_element_type=jnp.float32)
                      + bc2_ref[...])


def _ln(x, g, b, eps=1e-5):
    m = jnp.mean(x, axis=-1, keepdims=True)
    xc = x - m
    v = jnp.mean(xc * xc, axis=-1, keepdims=True)
    return xc * jax.lax.rsqrt(v + eps) * g + b


def _out_kernel(ids_ref, feat_ref, h_ref,
                Wf1_ref, bf1_ref, Wf2_ref, bf2_ref,
                ln1g_ref, ln1b_ref, ln2g_ref, ln2b_ref,
                out_ref, *, nb, c):
    idsv = ids_ref[0, 0, :]
    Of = (jax.lax.broadcasted_iota(jnp.int32, (nb, c), 0)
          == idsv[None, :]).astype(jnp.float32)
    upd = jax.lax.dot_general(Of, h_ref[...], (((0,), (0,)), ((), ())),
                              preferred_element_type=jnp.float32)
    u1 = _ln(feat_ref[...] + upd, ln1g_ref[...], ln1b_ref[...])
    f1 = jnp.maximum(
        jnp.dot(u1.astype(jnp.bfloat16), Wf1_ref[...],
                preferred_element_type=jnp.float32)
        + bf1_ref[...], 0.0)
    ffn = (jnp.dot(f1.astype(jnp.bfloat16), Wf2_ref[...],
                   preferred_element_type=jnp.float32)
           + bf2_ref[...])
    out_ref[...] = _ln(u1 + ffn, ln2g_ref[...], ln2b_ref[...])


def kernel(atom_features, atom_positions, block_features, block_id,
           centers, widths, Wg, bg, Wq, bq, Wk, bk, Wv, bv,
           Wc1, bc1, Wc2, bc2, Wf1, bf1, Wf2, bf2,
           ln1_g, ln1_b, ln2_g, ln2_b):
    n, h = atom_features.shape
    nb = block_features.shape[0]
    rbf_dim = centers.shape[0]
    hq = Wg.shape[1]
    C = 2048
    nsteps = n // C

    ids = block_id.astype(jnp.int32)
    ids_full = ids.reshape(1, 1, n)
    ids_chunked = ids.reshape(nsteps, 1, C)
    posT = atom_positions.T

    centers_col = centers.reshape(rbf_dim, 1).astype(jnp.float32)
    inv2w2_col = (1.0 / (2.0 * jnp.square(widths))).reshape(rbf_dim, 1)
    row = lambda v: v.reshape(1, -1)

    centroids = pl.pallas_call(
        functools.partial(_centroid_kernel, nb=nb, n=n),
        out_shape=jax.ShapeDtypeStruct((3, nb), jnp.float32),
    )(ids_full, posT)

    Wk_top, Wk_bot = Wk[:h].astype(jnp.bfloat16), Wk[h:].astype(jnp.bfloat16)
    Wv_top, Wv_bot = Wv[:h].astype(jnp.bfloat16), Wv[h:].astype(jnp.bfloat16)
    Wf1_bf = Wf1.astype(jnp.bfloat16)
    Wf2_bf = Wf2.astype(jnp.bfloat16)
    WgT = Wg.T
    bg_col = bg.reshape(hq, 1)

    full = lambda shape: pl.BlockSpec(shape, lambda i: (0,) * len(shape))
    h_blocks = pl.pallas_call(
        functools.partial(_attn_kernel, nb=nb, c=C, nsteps=nsteps),
        grid=(nsteps,),
        in_specs=[
            pl.BlockSpec((1, 1, C), lambda i: (i, 0, 0)),
            pl.BlockSpec((C, h), lambda i: (i, 0)),
            pl.BlockSpec((3, C), lambda i: (0, i)),
            full((3, nb)),
            full((nb, h)),
            full((rbf_dim, 1)),
            full((rbf_dim, 1)),
            full((hq, rbf_dim)),
            full((hq, 1)),
            full((h, h)),
            full((1, h)),
            full((h, h)),
            full((hq, h)),
            full((1, h)),
            full((h, h)),
            full((hq, h)),
            full((1, h)),
            full((h, h)),
            full((1, h)),
            full((h, h)),
            full((1, h)),
        ],
        out_specs=full((nb, h)),
        out_shape=jax.ShapeDtypeStruct((nb, h), jnp.float32),
        scratch_shapes=[
            pltpu.VMEM((nb, h), jnp.float32),
            pltpu.VMEM((nb, 1), jnp.float32),
            pltpu.VMEM((nb, 1), jnp.float32),
            pltpu.VMEM((nb, h), jnp.float32),
        ],
    )(ids_chunked, atom_features, posT, centroids, block_features,
      centers_col, inv2w2_col, WgT, bg_col, Wq, row(bq),
      Wk_top, Wk_bot, row(bk), Wv_top, Wv_bot, row(bv),
      Wc1, row(bc1), Wc2, row(bc2))

    out = pl.pallas_call(
        functools.partial(_out_kernel, nb=nb, c=C),
        grid=(nsteps,),
        in_specs=[
            pl.BlockSpec((1, 1, C), lambda i: (i, 0, 0)),
            pl.BlockSpec((C, h), lambda i: (i, 0)),
            full((nb, h)),
            full((h, 2 * h)),
            full((1, 2 * h)),
            full((2 * h, h)),
            full((1, h)),
            full((1, h)),
            full((1, h)),
            full((1, h)),
            full((1, h)),
        ],
        out_specs=pl.BlockSpec((C, h), lambda i: (i, 0)),
        out_shape=jax.ShapeDtypeStruct((n, h), jnp.float32),
    )(ids_chunked, atom_features, h_blocks,
      Wf1_bf, row(bf1), Wf2_bf, row(bf2),
      row(ln1_g), row(ln1_b), row(ln2_g), row(ln2_b))

    return out


# speedup vs baseline: 11.3448x; 1.1118x over previous
import functools
import math

import jax
import jax.numpy as jnp
from jax.experimental import pallas as pl
from jax.experimental.pallas import tpu as pltpu

H = 128
NEG = -1e30


def _centroid_kernel(ids_ref, posT_ref, cent_ref, *, nb, n):
    idsv = ids_ref[0, 0, :]
    blk = jax.lax.broadcasted_iota(jnp.int32, (nb, n), 0)
    O = (blk == idsv[None, :]).astype(jnp.float32)
    sums = jax.lax.dot_general(posT_ref[...], O, (((1,), (1,)), ((), ())),
                               preferred_element_type=jnp.float32)
    cnt = jax.lax.dot_general(jnp.ones((1, n), jnp.float32), O,
                              (((1,), (1,)), ((), ())),
                              preferred_element_type=jnp.float32)
    cent_ref[...] = sums / jnp.maximum(cnt, 1.0)


def _attn_kernel(ids_ref, feat_ref, posT_ref, cent_ref, bfeat_ref,
                 centers_ref, inv2w2_ref, WgT_ref, bg_ref,
                 Wq_ref, bq_ref, Wkt_ref, Wkb_ref, bk_ref,
                 Wvt_ref, Wvb_ref, bv_ref,
                 Wc1_ref, bc1_ref, Wc2_ref, bc2_ref,
                 h_ref, q_scr, m_scr, s_scr, c_scr, *, nb, c, nsteps):
    i = pl.program_id(0)

    @pl.when(i == 0)
    def _init():
        q_scr[...] = (jnp.dot(bfeat_ref[...], Wq_ref[...],
                              preferred_element_type=jnp.float32)
                      + bq_ref[...])
        m_scr[...] = jnp.full((nb, 1), NEG, jnp.float32)
        s_scr[...] = jnp.zeros((nb, 1), jnp.float32)
        c_scr[...] = jnp.zeros((nb, H), jnp.float32)

    idsv = ids_ref[0, 0, :]
    O = (jax.lax.broadcasted_iota(jnp.int32, (nb, c), 0)
         == idsv[None, :])
    Of = O.astype(jnp.float32)

    cent_g = jax.lax.dot_general(cent_ref[...], Of, (((1,), (0,)), ((), ())),
                                 preferred_element_type=jnp.float32)
    rel = posT_ref[...] - cent_g
    d = jnp.sqrt(jnp.sum(rel * rel, axis=0, keepdims=True))
    rbfT = jnp.exp(-jnp.square(d - centers_ref[...]) * inv2w2_ref[...])
    geomT = (jnp.dot(WgT_ref[...], rbfT, preferred_element_type=jnp.float32)
             + bg_ref[...]).astype(jnp.bfloat16)

    feats = feat_ref[...].astype(jnp.bfloat16)
    K = (jnp.dot(feats, Wkt_ref[...], preferred_element_type=jnp.float32)
         + jax.lax.dot_general(geomT, Wkb_ref[...], (((0,), (0,)), ((), ())),
                               preferred_element_type=jnp.float32)
         + bk_ref[...])
    V = (jnp.dot(feats, Wvt_ref[...], preferred_element_type=jnp.float32)
         + jax.lax.dot_general(geomT, Wvb_ref[...], (((0,), (0,)), ((), ())),
                               preferred_element_type=jnp.float32)
         + bv_ref[...])

    S = jax.lax.dot_general(q_scr[...], K, (((1,), (1,)), ((), ())),
                            preferred_element_type=jnp.float32)
    S = S * (1.0 / math.sqrt(H))
    Sm = jnp.where(O, S, NEG)
    m_old = m_scr[...]
    m_new = jnp.maximum(m_old, jnp.max(Sm, axis=1, keepdims=True))
    alpha = jnp.exp(m_old - m_new)
    e = jnp.where(O, jnp.exp(S - m_new), 0.0)
    m_scr[...] = m_new
    s_scr[...] = s_scr[...] * alpha + jnp.sum(e, axis=1, keepdims=True)
    c_scr[...] = (c_scr[...] * alpha
                  + jnp.dot(e, V, preferred_element_type=jnp.float32))

    @pl.when(i == nsteps - 1)
    def _finish():
        s = s_scr[...]
        ctx = c_scr[...] / jnp.where(s > 0.0, s, 1.0)
        h1 = jnp.maximum(
            jnp.dot(ctx, Wc1_ref[...], preferred_element_type=jnp.float32)
            + bc1_ref[...], 0.0)
        h_ref[...] = (jnp.dot(h1, Wc2_ref[...],
                              preferred_element_type=jnp.float32)
                      + bc2_ref[...])


def _ln(x, g, b, eps=1e-5):
    m = jnp.mean(x, axis=-1, keepdims=True)
    xc = x - m
    v = jnp.mean(xc * xc, axis=-1, keepdims=True)
    return xc * jax.lax.rsqrt(v + eps) * g + b


def _out_kernel(ids_ref, feat_ref, h_ref,
                Wf1_ref, bf1_ref, Wf2_ref, bf2_ref,
                ln1g_ref, ln1b_ref, ln2g_ref, ln2b_ref,
                out_ref, *, nb, c):
    idsv = ids_ref[0, 0, :]
    Of = (jax.lax.broadcasted_iota(jnp.int32, (nb, c), 0)
          == idsv[None, :]).astype(jnp.float32)
    upd = jax.lax.dot_general(Of, h_ref[...], (((0,), (0,)), ((), ())),
                              preferred_element_type=jnp.float32)
    u1 = _ln(feat_ref[...] + upd, ln1g_ref[...], ln1b_ref[...])
    f1 = jnp.maximum(
        jnp.dot(u1.astype(jnp.bfloat16), Wf1_ref[...],
                preferred_element_type=jnp.float32)
        + bf1_ref[...], 0.0)
    ffn = (jnp.dot(f1.astype(jnp.bfloat16), Wf2_ref[...],
                   preferred_element_type=jnp.float32)
           + bf2_ref[...])
    out_ref[...] = _ln(u1 + ffn, ln2g_ref[...], ln2b_ref[...])


def kernel(atom_features, atom_positions, block_features, block_id,
           centers, widths, Wg, bg, Wq, bq, Wk, bk, Wv, bv,
           Wc1, bc1, Wc2, bc2, Wf1, bf1, Wf2, bf2,
           ln1_g, ln1_b, ln2_g, ln2_b):
    n, h = atom_features.shape
    nb = block_features.shape[0]
    rbf_dim = centers.shape[0]
    hq = Wg.shape[1]
    C = 4096
    nsteps = n // C

    ids = block_id.astype(jnp.int32)
    ids_full = ids.reshape(1, 1, n)
    ids_chunked = ids.reshape(nsteps, 1, C)
    posT = atom_positions.T

    centers_col = centers.reshape(rbf_dim, 1).astype(jnp.float32)
    inv2w2_col = (1.0 / (2.0 * jnp.square(widths))).reshape(rbf_dim, 1)
    row = lambda v: v.reshape(1, -1)

    centroids = pl.pallas_call(
        functools.partial(_centroid_kernel, nb=nb, n=n),
        out_shape=jax.ShapeDtypeStruct((3, nb), jnp.float32),
    )(ids_full, posT)

    Wk_top, Wk_bot = Wk[:h].astype(jnp.bfloat16), Wk[h:].astype(jnp.bfloat16)
    Wv_top, Wv_bot = Wv[:h].astype(jnp.bfloat16), Wv[h:].astype(jnp.bfloat16)
    Wf1_bf = Wf1.astype(jnp.bfloat16)
    Wf2_bf = Wf2.astype(jnp.bfloat16)
    WgT = Wg.T
    bg_col = bg.reshape(hq, 1)

    full = lambda shape: pl.BlockSpec(shape, lambda i: (0,) * len(shape))
    h_blocks = pl.pallas_call(
        functools.partial(_attn_kernel, nb=nb, c=C, nsteps=nsteps),
        grid=(nsteps,),
        in_specs=[
            pl.BlockSpec((1, 1, C), lambda i: (i, 0, 0)),
            pl.BlockSpec((C, h), lambda i: (i, 0)),
            pl.BlockSpec((3, C), lambda i: (0, i)),
            full((3, nb)),
            full((nb, h)),
            full((rbf_dim, 1)),
            full((rbf_dim, 1)),
            full((hq, rbf_dim)),
            full((hq, 1)),
            full((h, h)),
            full((1, h)),
            full((h, h)),
            full((hq, h)),
            full((1, h)),
            full((h, h)),
            full((hq, h)),
            full((1, h)),
            full((h, h)),
            full((1, h)),
            full((h, h)),
            full((1, h)),
        ],
        out_specs=full((nb, h)),
        out_shape=jax.ShapeDtypeStruct((nb, h), jnp.float32),
        scratch_shapes=[
            pltpu.VMEM((nb, h), jnp.float32),
            pltpu.VMEM((nb, 1), jnp.float32),
            pltpu.VMEM((nb, 1), jnp.float32),
            pltpu.VMEM((nb, h), jnp.float32),
        ],
    )(ids_chunked, atom_features, posT, centroids, block_features,
      centers_col, inv2w2_col, WgT, bg_col, Wq, row(bq),
      Wk_top, Wk_bot, row(bk), Wv_top, Wv_bot, row(bv),
      Wc1, row(bc1), Wc2, row(bc2))

    out = pl.pallas_call(
        functools.partial(_out_kernel, nb=nb, c=C),
        grid=(nsteps,),
        in_specs=[
            pl.BlockSpec((1, 1, C), lambda i: (i, 0, 0)),
            pl.BlockSpec((C, h), lambda i: (i, 0)),
            full((nb, h)),
            full((h, 2 * h)),
            full((1, 2 * h)),
            full((2 * h, h)),
            full((1, h)),
            full((1, h)),
            full((1, h)),
            full((1, h)),
            full((1, h)),
        ],
        out_specs=pl.BlockSpec((C, h), lambda i: (i, 0)),
        out_shape=jax.ShapeDtypeStruct((n, h), jnp.float32),
    )(ids_chunked, atom_features, h_blocks,
      Wf1_bf, row(bf1), Wf2_bf, row(bf2),
      row(ln1_g), row(ln1_b), row(ln2_g), row(ln2_b))

    return out


# speedup vs baseline: 11.7955x; 1.0397x over previous
import functools
import math

import jax
import jax.numpy as jnp
from jax.experimental import pallas as pl
from jax.experimental.pallas import tpu as pltpu

H = 128
NEG = -1e30


def _centroid_kernel(ids_ref, posT_ref, cent_ref, *, nb, n):
    idsv = ids_ref[0, 0, :]
    blk = jax.lax.broadcasted_iota(jnp.int32, (nb, n), 0)
    O = (blk == idsv[None, :]).astype(jnp.float32)
    sums = jax.lax.dot_general(posT_ref[...], O, (((1,), (1,)), ((), ())),
                               preferred_element_type=jnp.float32)
    cnt = jax.lax.dot_general(jnp.ones((1, n), jnp.float32), O,
                              (((1,), (1,)), ((), ())),
                              preferred_element_type=jnp.float32)
    cent_ref[...] = sums / jnp.maximum(cnt, 1.0)


def _attn_kernel(ids_ref, feat_ref, posT_ref, cent_ref, bfeat_ref,
                 centers_ref, inv2w2_ref, WgT_ref, bg_ref,
                 Wq_ref, bq_ref, Wkt_ref, Wkb_ref, bk_ref,
                 Wvt_ref, Wvb_ref, bv_ref,
                 Wc1_ref, bc1_ref, Wc2_ref, bc2_ref,
                 h_ref, q_scr, m_scr, s_scr, c_scr, *, nb, c, nsteps):
    i = pl.program_id(0)

    @pl.when(i == 0)
    def _init():
        q_scr[...] = (jnp.dot(bfeat_ref[...], Wq_ref[...],
                              preferred_element_type=jnp.float32)
                      + bq_ref[...])
        m_scr[...] = jnp.full((nb, 1), NEG, jnp.float32)
        s_scr[...] = jnp.zeros((nb, 1), jnp.float32)
        c_scr[...] = jnp.zeros((nb, H), jnp.float32)

    idsv = ids_ref[0, 0, :]
    O = (jax.lax.broadcasted_iota(jnp.int32, (nb, c), 0)
         == idsv[None, :])
    Of = O.astype(jnp.float32)

    cent_g = jax.lax.dot_general(cent_ref[...], Of, (((1,), (0,)), ((), ())),
                                 preferred_element_type=jnp.float32)
    rel = posT_ref[...] - cent_g
    d = jnp.sqrt(jnp.sum(rel * rel, axis=0, keepdims=True))
    rbfT = jnp.exp(-jnp.square(d - centers_ref[...]) * inv2w2_ref[...])
    geomT = (jnp.dot(WgT_ref[...], rbfT, preferred_element_type=jnp.float32)
             + bg_ref[...]).astype(jnp.bfloat16)

    feats = feat_ref[...].astype(jnp.bfloat16)
    K = (jnp.dot(feats, Wkt_ref[...], preferred_element_type=jnp.float32)
         + jax.lax.dot_general(geomT, Wkb_ref[...], (((0,), (0,)), ((), ())),
                               preferred_element_type=jnp.float32)
         + bk_ref[...])
    V = (jnp.dot(feats, Wvt_ref[...], preferred_element_type=jnp.float32)
         + jax.lax.dot_general(geomT, Wvb_ref[...], (((0,), (0,)), ((), ())),
                               preferred_element_type=jnp.float32)
         + bv_ref[...])

    S = jax.lax.dot_general(q_scr[...], K, (((1,), (1,)), ((), ())),
                            preferred_element_type=jnp.float32)
    S = S * (1.0 / math.sqrt(H))
    Sm = jnp.where(O, S, NEG)
    m_old = m_scr[...]
    m_new = jnp.maximum(m_old, jnp.max(Sm, axis=1, keepdims=True))
    alpha = jnp.exp(m_old - m_new)
    e = jnp.where(O, jnp.exp(S - m_new), 0.0)
    m_scr[...] = m_new
    s_scr[...] = s_scr[...] * alpha + jnp.sum(e, axis=1, keepdims=True)
    c_scr[...] = (c_scr[...] * alpha
                  + jnp.dot(e, V, preferred_element_type=jnp.float32))

    @pl.when(i == nsteps - 1)
    def _finish():
        s = s_scr[...]
        ctx = c_scr[...] / jnp.where(s > 0.0, s, 1.0)
        h1 = jnp.maximum(
            jnp.dot(ctx, Wc1_ref[...], preferred_element_type=jnp.float32)
            + bc1_ref[...], 0.0)
        h_ref[...] = (jnp.dot(h1, Wc2_ref[...],
                              preferred_element_type=jnp.float32)
                      + bc2_ref[...])


def _ln(x, g, b, eps=1e-5):
    m = jnp.mean(x, axis=-1, keepdims=True)
    xc = x - m
    v = jnp.mean(xc * xc, axis=-1, keepdims=True)
    return xc * jax.lax.rsqrt(v + eps) * g + b


def _out_kernel(ids_ref, feat_ref, h_ref,
                Wf1_ref, bf1_ref, Wf2_ref, bf2_ref,
                ln1g_ref, ln1b_ref, ln2g_ref, ln2b_ref,
                out_ref, *, nb, c):
    idsv = ids_ref[0, 0, :]
    Of = (jax.lax.broadcasted_iota(jnp.int32, (nb, c), 0)
          == idsv[None, :]).astype(jnp.float32)
    upd = jax.lax.dot_general(Of, h_ref[...], (((0,), (0,)), ((), ())),
                              preferred_element_type=jnp.float32)
    u1 = _ln(feat_ref[...] + upd, ln1g_ref[...], ln1b_ref[...])
    f1 = jnp.maximum(
        jnp.dot(u1.astype(jnp.bfloat16), Wf1_ref[...],
                preferred_element_type=jnp.float32)
        + bf1_ref[...], 0.0)
    ffn = (jnp.dot(f1.astype(jnp.bfloat16), Wf2_ref[...],
                   preferred_element_type=jnp.float32)
           + bf2_ref[...])
    out_ref[...] = _ln(u1 + ffn, ln2g_ref[...], ln2b_ref[...])


def kernel(atom_features, atom_positions, block_features, block_id,
           centers, widths, Wg, bg, Wq, bq, Wk, bk, Wv, bv,
           Wc1, bc1, Wc2, bc2, Wf1, bf1, Wf2, bf2,
           ln1_g, ln1_b, ln2_g, ln2_b):
    n, h = atom_features.shape
    nb = block_features.shape[0]
    rbf_dim = centers.shape[0]
    hq = Wg.shape[1]
    C = 8192
    nsteps = n // C

    ids = block_id.astype(jnp.int32)
    ids_full = ids.reshape(1, 1, n)
    ids_chunked = ids.reshape(nsteps, 1, C)
    posT = atom_positions.T

    centers_col = centers.reshape(rbf_dim, 1).astype(jnp.float32)
    inv2w2_col = (1.0 / (2.0 * jnp.square(widths))).reshape(rbf_dim, 1)
    row = lambda v: v.reshape(1, -1)

    centroids = pl.pallas_call(
        functools.partial(_centroid_kernel, nb=nb, n=n),
        out_shape=jax.ShapeDtypeStruct((3, nb), jnp.float32),
    )(ids_full, posT)

    Wk_top, Wk_bot = Wk[:h].astype(jnp.bfloat16), Wk[h:].astype(jnp.bfloat16)
    Wv_top, Wv_bot = Wv[:h].astype(jnp.bfloat16), Wv[h:].astype(jnp.bfloat16)
    Wf1_bf = Wf1.astype(jnp.bfloat16)
    Wf2_bf = Wf2.astype(jnp.bfloat16)
    WgT = Wg.T
    bg_col = bg.reshape(hq, 1)

    full = lambda shape: pl.BlockSpec(shape, lambda i: (0,) * len(shape))
    h_blocks = pl.pallas_call(
        functools.partial(_attn_kernel, nb=nb, c=C, nsteps=nsteps),
        grid=(nsteps,),
        in_specs=[
            pl.BlockSpec((1, 1, C), lambda i: (i, 0, 0)),
            pl.BlockSpec((C, h), lambda i: (i, 0)),
            pl.BlockSpec((3, C), lambda i: (0, i)),
            full((3, nb)),
            full((nb, h)),
            full((rbf_dim, 1)),
            full((rbf_dim, 1)),
            full((hq, rbf_dim)),
            full((hq, 1)),
            full((h, h)),
            full((1, h)),
            full((h, h)),
            full((hq, h)),
            full((1, h)),
            full((h, h)),
            full((hq, h)),
            full((1, h)),
            full((h, h)),
            full((1, h)),
            full((h, h)),
            full((1, h)),
        ],
        out_specs=full((nb, h)),
        out_shape=jax.ShapeDtypeStruct((nb, h), jnp.float32),
        scratch_shapes=[
            pltpu.VMEM((nb, h), jnp.float32),
            pltpu.VMEM((nb, 1), jnp.float32),
            pltpu.VMEM((nb, 1), jnp.float32),
            pltpu.VMEM((nb, h), jnp.float32),
        ],
    )(ids_chunked, atom_features, posT, centroids, block_features,
      centers_col, inv2w2_col, WgT, bg_col, Wq, row(bq),
      Wk_top, Wk_bot, row(bk), Wv_top, Wv_bot, row(bv),
      Wc1, row(bc1), Wc2, row(bc2))

    out = pl.pallas_call(
        functools.partial(_out_kernel, nb=nb, c=C),
        grid=(nsteps,),
        in_specs=[
            pl.BlockSpec((1, 1, C), lambda i: (i, 0, 0)),
            pl.BlockSpec((C, h), lambda i: (i, 0)),
            full((nb, h)),
            full((h, 2 * h)),
            full((1, 2 * h)),
            full((2 * h, h)),
            full((1, h)),
            full((1, h)),
            full((1, h)),
            full((1, h)),
            full((1, h)),
        ],
        out_specs=pl.BlockSpec((C, h), lambda i: (i, 0)),
        out_shape=jax.ShapeDtypeStruct((n, h), jnp.float32),
    )(ids_chunked, atom_features, h_blocks,
      Wf1_bf, row(bf1), Wf2_bf, row(bf2),
      row(ln1_g), row(ln1_b), row(ln2_g), row(ln2_b))

    return out


# speedup vs baseline: 12.0516x; 1.0217x over previous
import functools
import math

import jax
import jax.numpy as jnp
from jax.experimental import pallas as pl
from jax.experimental.pallas import tpu as pltpu

H = 128
NEG = -1e30


def _centroid_kernel(ids_ref, posT_ref, cent_ref, *, nb, n):
    idsv = ids_ref[0, 0, :]
    blk = jax.lax.broadcasted_iota(jnp.int32, (nb, n), 0)
    O = (blk == idsv[None, :]).astype(jnp.float32)
    sums = jax.lax.dot_general(posT_ref[...], O, (((1,), (1,)), ((), ())),
                               preferred_element_type=jnp.float32)
    cnt = jax.lax.dot_general(jnp.ones((1, n), jnp.float32), O,
                              (((1,), (1,)), ((), ())),
                              preferred_element_type=jnp.float32)
    cent_ref[...] = sums / jnp.maximum(cnt, 1.0)


def _attn_kernel(ids_ref, feat_ref, posT_ref, cent_ref, bfeat_ref,
                 centers_ref, inv2w2_ref, WgT_ref, bg_ref,
                 Wq_ref, bq_ref, Wkt_ref, Wkb_ref, bk_ref,
                 Wvt_ref, Wvb_ref, bv_ref,
                 Wc1_ref, bc1_ref, Wc2_ref, bc2_ref,
                 h_ref, q_scr, m_scr, s_scr, c_scr, *, nb, c, nsteps):
    i = pl.program_id(0)

    @pl.when(i == 0)
    def _init():
        q_scr[...] = (jnp.dot(bfeat_ref[...], Wq_ref[...],
                              preferred_element_type=jnp.float32)
                      + bq_ref[...])
        m_scr[...] = jnp.full((nb, 1), NEG, jnp.float32)
        s_scr[...] = jnp.zeros((nb, 1), jnp.float32)
        c_scr[...] = jnp.zeros((nb, H), jnp.float32)

    idsv = ids_ref[0, 0, :]
    O = (jax.lax.broadcasted_iota(jnp.int32, (nb, c), 0)
         == idsv[None, :])
    Of = O.astype(jnp.float32)

    cent_g = jax.lax.dot_general(cent_ref[...], Of, (((1,), (0,)), ((), ())),
                                 preferred_element_type=jnp.float32)
    rel = posT_ref[...] - cent_g
    d = jnp.sqrt(jnp.sum(rel * rel, axis=0, keepdims=True))
    rbfT = jnp.exp(-jnp.square(d - centers_ref[...]) * inv2w2_ref[...])
    geomT = (jnp.dot(WgT_ref[...], rbfT, preferred_element_type=jnp.float32)
             + bg_ref[...]).astype(jnp.bfloat16)

    feats = feat_ref[...].astype(jnp.bfloat16)
    K = (jnp.dot(feats, Wkt_ref[...], preferred_element_type=jnp.float32)
         + jax.lax.dot_general(geomT, Wkb_ref[...], (((0,), (0,)), ((), ())),
                               preferred_element_type=jnp.float32)
         + bk_ref[...])
    V = (jnp.dot(feats, Wvt_ref[...], preferred_element_type=jnp.float32)
         + jax.lax.dot_general(geomT, Wvb_ref[...], (((0,), (0,)), ((), ())),
                               preferred_element_type=jnp.float32)
         + bv_ref[...])

    S = jax.lax.dot_general(q_scr[...], K, (((1,), (1,)), ((), ())),
                            preferred_element_type=jnp.float32)
    S = S * (1.0 / math.sqrt(H))
    Sm = jnp.where(O, S, NEG)
    m_old = m_scr[...]
    m_new = jnp.maximum(m_old, jnp.max(Sm, axis=1, keepdims=True))
    alpha = jnp.exp(m_old - m_new)
    e = jnp.where(O, jnp.exp(S - m_new), 0.0)
    m_scr[...] = m_new
    s_scr[...] = s_scr[...] * alpha + jnp.sum(e, axis=1, keepdims=True)
    c_scr[...] = (c_scr[...] * alpha
                  + jnp.dot(e, V, preferred_element_type=jnp.float32))

    @pl.when(i == nsteps - 1)
    def _finish():
        s = s_scr[...]
        ctx = c_scr[...] / jnp.where(s > 0.0, s, 1.0)
        h1 = jnp.maximum(
            jnp.dot(ctx, Wc1_ref[...], preferred_element_type=jnp.float32)
            + bc1_ref[...], 0.0)
        h_ref[...] = (jnp.dot(h1, Wc2_ref[...],
                              preferred_element_type=jnp.float32)
                      + bc2_ref[...])


def _ln(x, g, b, eps=1e-5):
    m = jnp.mean(x, axis=-1, keepdims=True)
    xc = x - m
    v = jnp.mean(xc * xc, axis=-1, keepdims=True)
    return xc * jax.lax.rsqrt(v + eps) * g + b


def _ln_mxu(x, g, b, eps=1e-5):
    J = jnp.full((H, H), 1.0 / H, jnp.float32)
    m = jnp.dot(x, J, preferred_element_type=jnp.float32)
    xc = x - m
    v = jnp.dot(xc * xc, J, preferred_element_type=jnp.float32)
    return xc * jax.lax.rsqrt(v + eps) * g + b


def _out_kernel(ids_ref, feat_ref, h_ref,
                Wf1_ref, bf1_ref, Wf2_ref, bf2_ref,
                ln1g_ref, ln1b_ref, ln2g_ref, ln2b_ref,
                out_ref, *, nb, c):
    idsv = ids_ref[0, 0, :]
    Of = (jax.lax.broadcasted_iota(jnp.int32, (nb, c), 0)
          == idsv[None, :]).astype(jnp.float32)
    upd = jax.lax.dot_general(Of, h_ref[...], (((0,), (0,)), ((), ())),
                              preferred_element_type=jnp.float32)
    u1 = _ln_mxu(feat_ref[...] + upd, ln1g_ref[...], ln1b_ref[...])
    f1 = jnp.maximum(
        jnp.dot(u1.astype(jnp.bfloat16), Wf1_ref[...],
                preferred_element_type=jnp.float32)
        + bf1_ref[...], 0.0)
    ffn = (jnp.dot(f1.astype(jnp.bfloat16), Wf2_ref[...],
                   preferred_element_type=jnp.float32)
           + bf2_ref[...])
    out_ref[...] = _ln_mxu(u1 + ffn, ln2g_ref[...], ln2b_ref[...])


def kernel(atom_features, atom_positions, block_features, block_id,
           centers, widths, Wg, bg, Wq, bq, Wk, bk, Wv, bv,
           Wc1, bc1, Wc2, bc2, Wf1, bf1, Wf2, bf2,
           ln1_g, ln1_b, ln2_g, ln2_b):
    n, h = atom_features.shape
    nb = block_features.shape[0]
    rbf_dim = centers.shape[0]
    hq = Wg.shape[1]
    C = 8192
    nsteps = n // C

    ids = block_id.astype(jnp.int32)
    ids_full = ids.reshape(1, 1, n)
    ids_chunked = ids.reshape(nsteps, 1, C)
    posT = atom_positions.T

    centers_col = centers.reshape(rbf_dim, 1).astype(jnp.float32)
    inv2w2_col = (1.0 / (2.0 * jnp.square(widths))).reshape(rbf_dim, 1)
    row = lambda v: v.reshape(1, -1)

    centroids = pl.pallas_call(
        functools.partial(_centroid_kernel, nb=nb, n=n),
        out_shape=jax.ShapeDtypeStruct((3, nb), jnp.float32),
    )(ids_full, posT)

    Wk_top, Wk_bot = Wk[:h].astype(jnp.bfloat16), Wk[h:].astype(jnp.bfloat16)
    Wv_top, Wv_bot = Wv[:h].astype(jnp.bfloat16), Wv[h:].astype(jnp.bfloat16)
    Wf1_bf = Wf1.astype(jnp.bfloat16)
    Wf2_bf = Wf2.astype(jnp.bfloat16)
    WgT = Wg.T
    bg_col = bg.reshape(hq, 1)

    full = lambda shape: pl.BlockSpec(shape, lambda i: (0,) * len(shape))
    h_blocks = pl.pallas_call(
        functools.partial(_attn_kernel, nb=nb, c=C, nsteps=nsteps),
        grid=(nsteps,),
        in_specs=[
            pl.BlockSpec((1, 1, C), lambda i: (i, 0, 0)),
            pl.BlockSpec((C, h), lambda i: (i, 0)),
            pl.BlockSpec((3, C), lambda i: (0, i)),
            full((3, nb)),
            full((nb, h)),
            full((rbf_dim, 1)),
            full((rbf_dim, 1)),
            full((hq, rbf_dim)),
            full((hq, 1)),
            full((h, h)),
            full((1, h)),
            full((h, h)),
            full((hq, h)),
            full((1, h)),
            full((h, h)),
            full((hq, h)),
            full((1, h)),
            full((h, h)),
            full((1, h)),
            full((h, h)),
            full((1, h)),
        ],
        out_specs=full((nb, h)),
        out_shape=jax.ShapeDtypeStruct((nb, h), jnp.float32),
        scratch_shapes=[
            pltpu.VMEM((nb, h), jnp.float32),
            pltpu.VMEM((nb, 1), jnp.float32),
            pltpu.VMEM((nb, 1), jnp.float32),
            pltpu.VMEM((nb, h), jnp.float32),
        ],
    )(ids_chunked, atom_features, posT, centroids, block_features,
      centers_col, inv2w2_col, WgT, bg_col, Wq, row(bq),
      Wk_top, Wk_bot, row(bk), Wv_top, Wv_bot, row(bv),
      Wc1, row(bc1), Wc2, row(bc2))

    out = pl.pallas_call(
        functools.partial(_out_kernel, nb=nb, c=C),
        grid=(nsteps,),
        in_specs=[
            pl.BlockSpec((1, 1, C), lambda i: (i, 0, 0)),
            pl.BlockSpec((C, h), lambda i: (i, 0)),
            full((nb, h)),
            full((h, 2 * h)),
            full((1, 2 * h)),
            full((2 * h, h)),
            full((1, h)),
            full((1, h)),
            full((1, h)),
            full((1, h)),
            full((1, h)),
        ],
        out_specs=pl.BlockSpec((C, h), lambda i: (i, 0)),
        out_shape=jax.ShapeDtypeStruct((n, h), jnp.float32),
    )(ids_chunked, atom_features, h_blocks,
      Wf1_bf, row(bf1), Wf2_bf, row(bf2),
      row(ln1_g), row(ln1_b), row(ln2_g), row(ln2_b))

    return out


# speedup vs baseline: 12.2030x; 1.0126x over previous
import functools
import math

import jax
import jax.numpy as jnp
from jax.experimental import pallas as pl
from jax.experimental.pallas import tpu as pltpu

H = 128
NEG = -1e30


def _ln_mxu(x, g, b, eps=1e-5):
    J = jnp.full((H, H), 1.0 / H, jnp.float32)
    m = jnp.dot(x, J, preferred_element_type=jnp.float32)
    xc = x - m
    v = jnp.dot(xc * xc, J, preferred_element_type=jnp.float32)
    return xc * jax.lax.rsqrt(v + eps) * g + b


def _fused_kernel(ids_a_ref, ids_b_ref, ids_c_ref,
                  pos_a_ref, pos_b_ref,
                  feat_b_ref, feat_c_ref, bfeat_ref,
                  centers_ref, inv2w2_ref, WgT_ref, bg_ref,
                  Wq_ref, bq_ref, Wkt_ref, Wkb_ref, bk_ref,
                  Wvt_ref, Wvb_ref, bv_ref,
                  Wc1_ref, bc1_ref, Wc2_ref, bc2_ref,
                  Wf1_ref, bf1_ref, Wf2_ref, bf2_ref,
                  ln1g_ref, ln1b_ref, ln2g_ref, ln2b_ref,
                  out_ref,
                  psum_scr, cnt_scr, cent_scr, q_scr,
                  m_scr, s_scr, c_scr, h_scr,
                  *, nb, c, nsteps):
    i = pl.program_id(0)

    @pl.when(i == 0)
    def _init():
        psum_scr[...] = jnp.zeros((3, nb), jnp.float32)
        cnt_scr[...] = jnp.zeros((1, nb), jnp.float32)
        q_scr[...] = (jnp.dot(bfeat_ref[...], Wq_ref[...],
                              preferred_element_type=jnp.float32)
                      + bq_ref[...])
        m_scr[...] = jnp.full((nb, 1), NEG, jnp.float32)
        s_scr[...] = jnp.zeros((nb, 1), jnp.float32)
        c_scr[...] = jnp.zeros((nb, H), jnp.float32)

    @pl.when(i < nsteps)
    def _phase_a():
        idsv = ids_a_ref[0, 0, :]
        Of = (jax.lax.broadcasted_iota(jnp.int32, (nb, c), 0)
              == idsv[None, :]).astype(jnp.float32)
        psum_scr[...] += jax.lax.dot_general(
            pos_a_ref[...], Of, (((1,), (1,)), ((), ())),
            preferred_element_type=jnp.float32)
        cnt_scr[...] += jax.lax.dot_general(
            jnp.ones((1, c), jnp.float32), Of, (((1,), (1,)), ((), ())),
            preferred_element_type=jnp.float32)

        @pl.when(i == nsteps - 1)
        def _fin_a():
            cent_scr[...] = psum_scr[...] / jnp.maximum(cnt_scr[...], 1.0)

    @pl.when((i >= nsteps) & (i < 2 * nsteps))
    def _phase_b():
        idsv = ids_b_ref[0, 0, :]
        O = (jax.lax.broadcasted_iota(jnp.int32, (nb, c), 0)
             == idsv[None, :])
        Of = O.astype(jnp.float32)

        cent_g = jax.lax.dot_general(cent_scr[...], Of,
                                     (((1,), (0,)), ((), ())),
                                     preferred_element_type=jnp.float32)
        rel = pos_b_ref[...] - cent_g
        d = jnp.sqrt(jnp.sum(rel * rel, axis=0, keepdims=True))
        rbfT = jnp.exp(-jnp.square(d - centers_ref[...]) * inv2w2_ref[...])
        geomT = (jnp.dot(WgT_ref[...], rbfT,
                         preferred_element_type=jnp.float32)
                 + bg_ref[...]).astype(jnp.bfloat16)

        feats = feat_b_ref[...].astype(jnp.bfloat16)
        K = (jnp.dot(feats, Wkt_ref[...], preferred_element_type=jnp.float32)
             + jax.lax.dot_general(geomT, Wkb_ref[...],
                                   (((0,), (0,)), ((), ())),
                                   preferred_element_type=jnp.float32)
             + bk_ref[...])
        V = (jnp.dot(feats, Wvt_ref[...], preferred_element_type=jnp.float32)
             + jax.lax.dot_general(geomT, Wvb_ref[...],
                                   (((0,), (0,)), ((), ())),
                                   preferred_element_type=jnp.float32)
             + bv_ref[...])

        S = jax.lax.dot_general(q_scr[...], K, (((1,), (1,)), ((), ())),
                                preferred_element_type=jnp.float32)
        S = S * (1.0 / math.sqrt(H))
        Sm = jnp.where(O, S, NEG)
        m_old = m_scr[...]
        m_new = jnp.maximum(m_old, jnp.max(Sm, axis=1, keepdims=True))
        alpha = jnp.exp(m_old - m_new)
        e = jnp.where(O, jnp.exp(S - m_new), 0.0)
        m_scr[...] = m_new
        s_scr[...] = s_scr[...] * alpha + jnp.sum(e, axis=1, keepdims=True)
        c_scr[...] = (c_scr[...] * alpha
                      + jnp.dot(e, V, preferred_element_type=jnp.float32))

        @pl.when(i == 2 * nsteps - 1)
        def _fin_b():
            s = s_scr[...]
            ctx = c_scr[...] / jnp.where(s > 0.0, s, 1.0)
            h1 = jnp.maximum(
                jnp.dot(ctx, Wc1_ref[...],
                        preferred_element_type=jnp.float32)
                + bc1_ref[...], 0.0)
            h_scr[...] = (jnp.dot(h1, Wc2_ref[...],
                                  preferred_element_type=jnp.float32)
                          + bc2_ref[...])

    @pl.when(i >= 2 * nsteps)
    def _phase_c():
        idsv = ids_c_ref[0, 0, :]
        Of = (jax.lax.broadcasted_iota(jnp.int32, (nb, c), 0)
              == idsv[None, :]).astype(jnp.float32)
        upd = jax.lax.dot_general(Of, h_scr[...], (((0,), (0,)), ((), ())),
                                  preferred_element_type=jnp.float32)
        u1 = _ln_mxu(feat_c_ref[...] + upd, ln1g_ref[...], ln1b_ref[...])
        f1 = jnp.maximum(
            jnp.dot(u1.astype(jnp.bfloat16), Wf1_ref[...],
                    preferred_element_type=jnp.float32)
            + bf1_ref[...], 0.0)
        ffn = (jnp.dot(f1.astype(jnp.bfloat16), Wf2_ref[...],
                       preferred_element_type=jnp.float32)
               + bf2_ref[...])
        out_ref[...] = _ln_mxu(u1 + ffn, ln2g_ref[...], ln2b_ref[...])


def kernel(atom_features, atom_positions, block_features, block_id,
           centers, widths, Wg, bg, Wq, bq, Wk, bk, Wv, bv,
           Wc1, bc1, Wc2, bc2, Wf1, bf1, Wf2, bf2,
           ln1_g, ln1_b, ln2_g, ln2_b):
    n, h = atom_features.shape
    nb = block_features.shape[0]
    rbf_dim = centers.shape[0]
    hq = Wg.shape[1]
    C = 8192
    nsteps = n // C

    ids = block_id.astype(jnp.int32)
    ids_chunked = ids.reshape(nsteps, 1, C)
    posT = atom_positions.T

    centers_col = centers.reshape(rbf_dim, 1).astype(jnp.float32)
    inv2w2_col = (1.0 / (2.0 * jnp.square(widths))).reshape(rbf_dim, 1)
    row = lambda v: v.reshape(1, -1)

    Wk_top, Wk_bot = Wk[:h].astype(jnp.bfloat16), Wk[h:].astype(jnp.bfloat16)
    Wv_top, Wv_bot = Wv[:h].astype(jnp.bfloat16), Wv[h:].astype(jnp.bfloat16)
    Wf1_bf = Wf1.astype(jnp.bfloat16)
    Wf2_bf = Wf2.astype(jnp.bfloat16)
    WgT = Wg.T
    bg_col = bg.reshape(hq, 1)

    last = nsteps - 1
    chunk_a = lambda i: (jnp.clip(i, 0, last), 0, 0)
    chunk_b = lambda i: (jnp.clip(i - nsteps, 0, last), 0, 0)
    chunk_c = lambda i: (jnp.clip(i - 2 * nsteps, 0, last), 0, 0)
    posm_a = lambda i: (0, jnp.clip(i, 0, last))
    posm_b = lambda i: (0, jnp.clip(i - nsteps, 0, last))
    featm_b = lambda i: (jnp.clip(i - nsteps, 0, last), 0)
    featm_c = lambda i: (jnp.clip(i - 2 * nsteps, 0, last), 0)
    full = lambda shape: pl.BlockSpec(shape, lambda i: (0,) * len(shape))

    out = pl.pallas_call(
        functools.partial(_fused_kernel, nb=nb, c=C, nsteps=nsteps),
        grid=(3 * nsteps,),
        in_specs=[
            pl.BlockSpec((1, 1, C), chunk_a),
            pl.BlockSpec((1, 1, C), chunk_b),
            pl.BlockSpec((1, 1, C), chunk_c),
            pl.BlockSpec((3, C), posm_a),
            pl.BlockSpec((3, C), posm_b),
            pl.BlockSpec((C, h), featm_b),
            pl.BlockSpec((C, h), featm_c),
            full((nb, h)),
            full((rbf_dim, 1)),
            full((rbf_dim, 1)),
            full((hq, rbf_dim)),
            full((hq, 1)),
            full((h, h)),
            full((1, h)),
            full((h, h)),
            full((hq, h)),
            full((1, h)),
            full((h, h)),
            full((hq, h)),
            full((1, h)),
            full((h, h)),
            full((1, h)),
            full((h, h)),
            full((1, h)),
            full((h, 2 * h)),
            full((1, 2 * h)),
            full((2 * h, h)),
            full((1, h)),
            full((1, h)),
            full((1, h)),
            full((1, h)),
            full((1, h)),
        ],
        out_specs=pl.BlockSpec((C, h), featm_c),
        out_shape=jax.ShapeDtypeStruct((n, h), jnp.float32),
        scratch_shapes=[
            pltpu.VMEM((3, nb), jnp.float32),
            pltpu.VMEM((1, nb), jnp.float32),
            pltpu.VMEM((3, nb), jnp.float32),
            pltpu.VMEM((nb, h), jnp.float32),
            pltpu.VMEM((nb, 1), jnp.float32),
            pltpu.VMEM((nb, 1), jnp.float32),
            pltpu.VMEM((nb, h), jnp.float32),
            pltpu.VMEM((nb, h), jnp.float32),
        ],
    )(ids_chunked, ids_chunked, ids_chunked,
      posT, posT,
      atom_features, atom_features, block_features,
      centers_col, inv2w2_col, WgT, bg_col,
      Wq, row(bq), Wk_top, Wk_bot, row(bk),
      Wv_top, Wv_bot, row(bv),
      Wc1, row(bc1), Wc2, row(bc2),
      Wf1_bf, row(bf1), Wf2_bf, row(bf2),
      row(ln1_g), row(ln1_b), row(ln2_g), row(ln2_b))

    return out


# speedup vs baseline: 12.3398x; 1.0112x over previous
import functools
import math

import jax
import jax.numpy as jnp
from jax.experimental import pallas as pl
from jax.experimental.pallas import tpu as pltpu

H = 128
NEG = -1e30


def _ln_mxu(x, g, b, eps=1e-5):
    J = jnp.full((H, H), 1.0 / H, jnp.float32)
    m = jnp.dot(x, J, preferred_element_type=jnp.float32)
    xc = x - m
    v = jnp.dot(xc * xc, J, preferred_element_type=jnp.float32)
    return xc * jax.lax.rsqrt(v + eps) * g + b


def _fused_kernel(ids_a_ref, ids_b_ref, ids_c_ref,
                  pos_a_ref, pos_b_ref,
                  feat_b_ref, bfeat_ref,
                  centers_ref, inv2w2_ref, WgT_ref, bg_ref,
                  Wq_ref, bq_ref, Wkt_ref, Wkb_ref, bk_ref,
                  Wvt_ref, Wvb_ref, bv_ref,
                  Wc1_ref, bc1_ref, Wc2_ref, bc2_ref,
                  Wf1_ref, bf1_ref, Wf2_ref, bf2_ref,
                  ln1g_ref, ln1b_ref, ln2g_ref, ln2b_ref,
                  out_ref,
                  psum_scr, cnt_scr, cent_scr, q_scr,
                  m_scr, s_scr, c_scr, h_scr, feat_scr,
                  *, nb, c, nsteps):
    i = pl.program_id(0)

    @pl.when(i == 0)
    def _init():
        psum_scr[...] = jnp.zeros((3, nb), jnp.float32)
        cnt_scr[...] = jnp.zeros((1, nb), jnp.float32)
        q_scr[...] = (jnp.dot(bfeat_ref[...], Wq_ref[...],
                              preferred_element_type=jnp.float32)
                      + bq_ref[...])
        m_scr[...] = jnp.full((nb, 1), NEG, jnp.float32)
        s_scr[...] = jnp.zeros((nb, 1), jnp.float32)
        c_scr[...] = jnp.zeros((nb, H), jnp.float32)

    @pl.when(i < nsteps)
    def _phase_a():
        idsv = ids_a_ref[0, 0, :]
        Of = (jax.lax.broadcasted_iota(jnp.int32, (nb, c), 0)
              == idsv[None, :]).astype(jnp.float32)
        psum_scr[...] += jax.lax.dot_general(
            pos_a_ref[...], Of, (((1,), (1,)), ((), ())),
            preferred_element_type=jnp.float32)
        cnt_scr[...] += jax.lax.dot_general(
            jnp.ones((1, c), jnp.float32), Of, (((1,), (1,)), ((), ())),
            preferred_element_type=jnp.float32)

        @pl.when(i == nsteps - 1)
        def _fin_a():
            cent_scr[...] = psum_scr[...] / jnp.maximum(cnt_scr[...], 1.0)

    @pl.when((i >= nsteps) & (i < 2 * nsteps))
    def _phase_b():
        idsv = ids_b_ref[0, 0, :]
        O = (jax.lax.broadcasted_iota(jnp.int32, (nb, c), 0)
             == idsv[None, :])
        Of = O.astype(jnp.float32)

        cent_g = jax.lax.dot_general(cent_scr[...], Of,
                                     (((1,), (0,)), ((), ())),
                                     preferred_element_type=jnp.float32)
        rel = pos_b_ref[...] - cent_g
        d = jnp.sqrt(jnp.sum(rel * rel, axis=0, keepdims=True))
        rbfT = jnp.exp(-jnp.square(d - centers_ref[...]) * inv2w2_ref[...])
        geomT = (jnp.dot(WgT_ref[...], rbfT,
                         preferred_element_type=jnp.float32)
                 + bg_ref[...]).astype(jnp.bfloat16)

        feats_f32 = feat_b_ref[...]
        feat_scr[pl.ds((i - nsteps) * c, c), :] = feats_f32
        feats = feats_f32.astype(jnp.bfloat16)
        K = (jnp.dot(feats, Wkt_ref[...], preferred_element_type=jnp.float32)
             + jax.lax.dot_general(geomT, Wkb_ref[...],
                                   (((0,), (0,)), ((), ())),
                                   preferred_element_type=jnp.float32)
             + bk_ref[...])
        V = (jnp.dot(feats, Wvt_ref[...], preferred_element_type=jnp.float32)
             + jax.lax.dot_general(geomT, Wvb_ref[...],
                                   (((0,), (0,)), ((), ())),
                                   preferred_element_type=jnp.float32)
             + bv_ref[...])

        S = jax.lax.dot_general(q_scr[...], K, (((1,), (1,)), ((), ())),
                                preferred_element_type=jnp.float32)
        S = S * (1.0 / math.sqrt(H))
        Sm = jnp.where(O, S, NEG)
        m_old = m_scr[...]
        m_new = jnp.maximum(m_old, jnp.max(Sm, axis=1, keepdims=True))
        alpha = jnp.exp(m_old - m_new)
        e = jnp.where(O, jnp.exp(S - m_new), 0.0)
        m_scr[...] = m_new
        s_scr[...] = s_scr[...] * alpha + jnp.sum(e, axis=1, keepdims=True)
        c_scr[...] = (c_scr[...] * alpha
                      + jnp.dot(e, V, preferred_element_type=jnp.float32))

        @pl.when(i == 2 * nsteps - 1)
        def _fin_b():
            s = s_scr[...]
            ctx = c_scr[...] / jnp.where(s > 0.0, s, 1.0)
            h1 = jnp.maximum(
                jnp.dot(ctx, Wc1_ref[...],
                        preferred_element_type=jnp.float32)
                + bc1_ref[...], 0.0)
            h_scr[...] = (jnp.dot(h1, Wc2_ref[...],
                                  preferred_element_type=jnp.float32)
                          + bc2_ref[...])

    @pl.when(i >= 2 * nsteps)
    def _phase_c():
        idsv = ids_c_ref[0, 0, :]
        Of = (jax.lax.broadcasted_iota(jnp.int32, (nb, c), 0)
              == idsv[None, :]).astype(jnp.float32)
        upd = jax.lax.dot_general(Of, h_scr[...], (((0,), (0,)), ((), ())),
                                  preferred_element_type=jnp.float32)
        feats = feat_scr[pl.ds((i - 2 * nsteps) * c, c), :]
        u1 = _ln_mxu(feats + upd, ln1g_ref[...], ln1b_ref[...])
        f1 = jnp.maximum(
            jnp.dot(u1.astype(jnp.bfloat16), Wf1_ref[...],
                    preferred_element_type=jnp.float32)
            + bf1_ref[...], 0.0)
        ffn = (jnp.dot(f1.astype(jnp.bfloat16), Wf2_ref[...],
                       preferred_element_type=jnp.float32)
               + bf2_ref[...])
        out_ref[...] = _ln_mxu(u1 + ffn, ln2g_ref[...], ln2b_ref[...])


def kernel(atom_features, atom_positions, block_features, block_id,
           centers, widths, Wg, bg, Wq, bq, Wk, bk, Wv, bv,
           Wc1, bc1, Wc2, bc2, Wf1, bf1, Wf2, bf2,
           ln1_g, ln1_b, ln2_g, ln2_b):
    n, h = atom_features.shape
    nb = block_features.shape[0]
    rbf_dim = centers.shape[0]
    hq = Wg.shape[1]
    C = 8192
    nsteps = n // C

    ids = block_id.astype(jnp.int32)
    ids_chunked = ids.reshape(nsteps, 1, C)
    posT = atom_positions.T

    centers_col = centers.reshape(rbf_dim, 1).astype(jnp.float32)
    inv2w2_col = (1.0 / (2.0 * jnp.square(widths))).reshape(rbf_dim, 1)
    row = lambda v: v.reshape(1, -1)

    Wk_top, Wk_bot = Wk[:h].astype(jnp.bfloat16), Wk[h:].astype(jnp.bfloat16)
    Wv_top, Wv_bot = Wv[:h].astype(jnp.bfloat16), Wv[h:].astype(jnp.bfloat16)
    Wf1_bf = Wf1.astype(jnp.bfloat16)
    Wf2_bf = Wf2.astype(jnp.bfloat16)
    WgT = Wg.T
    bg_col = bg.reshape(hq, 1)

    last = nsteps - 1
    chunk_a = lambda i: (jnp.clip(i, 0, last), 0, 0)
    chunk_b = lambda i: (jnp.clip(i - nsteps, 0, last), 0, 0)
    chunk_c = lambda i: (jnp.clip(i - 2 * nsteps, 0, last), 0, 0)
    posm_a = lambda i: (0, jnp.clip(i, 0, last))
    posm_b = lambda i: (0, jnp.clip(i - nsteps, 0, last))
    featm_b = lambda i: (jnp.clip(i - nsteps, 0, last), 0)
    featm_c = lambda i: (jnp.clip(i - 2 * nsteps, 0, last), 0)
    full = lambda shape: pl.BlockSpec(shape, lambda i: (0,) * len(shape))

    out = pl.pallas_call(
        functools.partial(_fused_kernel, nb=nb, c=C, nsteps=nsteps),
        grid=(3 * nsteps,),
        in_specs=[
            pl.BlockSpec((1, 1, C), chunk_a),
            pl.BlockSpec((1, 1, C), chunk_b),
            pl.BlockSpec((1, 1, C), chunk_c),
            pl.BlockSpec((3, C), posm_a),
            pl.BlockSpec((3, C), posm_b),
            pl.BlockSpec((C, h), featm_b),
            full((nb, h)),
            full((rbf_dim, 1)),
            full((rbf_dim, 1)),
            full((hq, rbf_dim)),
            full((hq, 1)),
            full((h, h)),
            full((1, h)),
            full((h, h)),
            full((hq, h)),
            full((1, h)),
            full((h, h)),
            full((hq, h)),
            full((1, h)),
            full((h, h)),
            full((1, h)),
            full((h, h)),
            full((1, h)),
            full((h, 2 * h)),
            full((1, 2 * h)),
            full((2 * h, h)),
            full((1, h)),
            full((1, h)),
            full((1, h)),
            full((1, h)),
            full((1, h)),
        ],
        out_specs=pl.BlockSpec((C, h), featm_c),
        out_shape=jax.ShapeDtypeStruct((n, h), jnp.float32),
        scratch_shapes=[
            pltpu.VMEM((3, nb), jnp.float32),
            pltpu.VMEM((1, nb), jnp.float32),
            pltpu.VMEM((3, nb), jnp.float32),
            pltpu.VMEM((nb, h), jnp.float32),
            pltpu.VMEM((nb, 1), jnp.float32),
            pltpu.VMEM((nb, 1), jnp.float32),
            pltpu.VMEM((nb, h), jnp.float32),
            pltpu.VMEM((nb, h), jnp.float32),
            pltpu.VMEM((n, h), jnp.float32),
        ],
    )(ids_chunked, ids_chunked, ids_chunked,
      posT, posT,
      atom_features, block_features,
      centers_col, inv2w2_col, WgT, bg_col,
      Wq, row(bq), Wk_top, Wk_bot, row(bk),
      Wv_top, Wv_bot, row(bv),
      Wc1, row(bc1), Wc2, row(bc2),
      Wf1_bf, row(bf1), Wf2_bf, row(bf2),
      row(ln1_g), row(ln1_b), row(ln2_g), row(ln2_b))

    return out
